# baseline (speedup 1.0000x reference)
import functools

import jax
import jax.numpy as jnp
from jax import lax
from jax.experimental import pallas as pl
from jax.experimental.pallas import tpu as pltpu
from jax.experimental.pallas import tpu_sc as plsc

_IDXW = 128


def kernel(inputs, table):
    B, S = inputs.shape
    V, D = table.shape
    n_rows = B * S
    idx2 = inputs.reshape(n_rows // _IDXW, _IDXW).astype(jnp.int32)

    info = plsc.get_sparse_core_info()
    NC, NS = info.num_cores, info.num_subcores
    NW = NC * NS
    rows_per_w = n_rows // NW
    CH = 1024
    n_sub = CH // _IDXW
    n_chunks = rows_per_w // CH

    mesh = plsc.VectorSubcoreMesh(core_axis_name="c", subcore_axis_name="s")

    @functools.partial(
        pl.kernel,
        mesh=mesh,
        out_type=jax.ShapeDtypeStruct((n_rows, D), jnp.float32),
        scratch_types=[
            pltpu.VMEM((n_sub, _IDXW), jnp.int32),
            pltpu.VMEM((CH, D), jnp.float32),
            pltpu.SemaphoreType.DMA,
        ],
        compiler_params=pltpu.CompilerParams(use_tc_tiling_on_sc=False),
    )
    def gather_kernel(idx_hbm, table_hbm, out_hbm, idx_v, rows_v, sem):
        wid = lax.axis_index("s") * NC + lax.axis_index("c")
        base = wid * rows_per_w
        base_row = wid * (rows_per_w // _IDXW)

        def body(i, carry):
            off = base + i * CH
            pltpu.sync_copy(idx_hbm.at[pl.ds(base_row + i * n_sub, n_sub)], idx_v)
            for j in range(n_sub):
                pltpu.async_copy(
                    table_hbm.at[idx_v.at[j]],
                    rows_v.at[pl.ds(j * _IDXW, _IDXW)],
                    sem,
                )
            for j in range(n_sub):
                pltpu.make_async_copy(
                    table_hbm.at[idx_v.at[j]],
                    rows_v.at[pl.ds(j * _IDXW, _IDXW)],
                    sem,
                ).wait()
            pltpu.sync_copy(rows_v, out_hbm.at[pl.ds(off, CH)])
            return carry

        lax.fori_loop(0, n_chunks, body, 0)

    out = gather_kernel(idx2, table)
    return out.reshape(B, S, D)


# speedup vs baseline: 1.0375x; 1.0375x over previous
import functools

import jax
import jax.numpy as jnp
from jax import lax
from jax.experimental import pallas as pl
from jax.experimental.pallas import tpu as pltpu
from jax.experimental.pallas import tpu_sc as plsc

_IDXW = 128


def kernel(inputs, table):
    B, S = inputs.shape
    V, D = table.shape
    n_rows = B * S
    idx2 = inputs.reshape(n_rows // _IDXW, _IDXW).astype(jnp.int32)
    table_w = jnp.pad(table, ((0, 0), (0, 128 - D)))

    info = plsc.get_sparse_core_info()
    NC, NS = info.num_cores, info.num_subcores
    NW = NC * NS
    rows_per_w = n_rows // NW
    CH = 256
    n_sub = CH // _IDXW
    n_chunks = rows_per_w // CH

    mesh = plsc.VectorSubcoreMesh(core_axis_name="c", subcore_axis_name="s")

    @functools.partial(
        pl.kernel,
        mesh=mesh,
        out_type=jax.ShapeDtypeStruct((n_rows, D), jnp.float32),
        scratch_types=[
            pltpu.VMEM((n_sub, _IDXW), jnp.int32),
            pltpu.VMEM((CH, 128), jnp.float32),
            pltpu.VMEM((CH, D), jnp.float32),
            pltpu.SemaphoreType.DMA,
        ],
    )
    def gather_kernel(idx_hbm, table_hbm, out_hbm, idx_v, rows_v, rows_c, sem):
        wid = lax.axis_index("s") * NC + lax.axis_index("c")
        base = wid * rows_per_w
        base_row = wid * (rows_per_w // _IDXW)

        def body(i, carry):
            off = base + i * CH
            pltpu.sync_copy(idx_hbm.at[pl.ds(base_row + i * n_sub, n_sub)], idx_v)
            for j in range(n_sub):
                pltpu.async_copy(
                    table_hbm.at[idx_v.at[j]],
                    rows_v.at[pl.ds(j * _IDXW, _IDXW)],
                    sem,
                )
            for j in range(n_sub):
                pltpu.make_async_copy(
                    table_hbm.at[idx_v.at[j]],
                    rows_v.at[pl.ds(j * _IDXW, _IDXW)],
                    sem,
                ).wait()

            def compact(r, c2):
                for k in range(D // 16):
                    rows_c[r, pl.ds(k * 16, 16)] = rows_v[r, pl.ds(k * 16, 16)]
                return c2

            lax.fori_loop(0, CH, compact, 0)
            pltpu.sync_copy(rows_c, out_hbm.at[pl.ds(off, CH)])
            return carry

        lax.fori_loop(0, n_chunks, body, 0)

    out = gather_kernel(idx2, table_w)
    return out.reshape(B, S, D)


# speedup vs baseline: 1.1210x; 1.0805x over previous
import functools

import jax
import jax.numpy as jnp
from jax import lax
from jax.experimental import pallas as pl
from jax.experimental.pallas import tpu as pltpu
from jax.experimental.pallas import tpu_sc as plsc


def kernel(inputs, table):
    B, S = inputs.shape
    V, D = table.shape
    n_rows = B * S
    idx_flat = inputs.reshape(n_rows).astype(jnp.int32)
    table_w = jnp.pad(table, ((0, 0), (0, 128 - D)))

    info = plsc.get_sparse_core_info()
    NC, NS = info.num_cores, info.num_subcores
    NW = NC * NS
    rows_per_w = n_rows // NW
    slabs_per_w = rows_per_w // S
    G0 = 128
    G1 = S - G0

    mesh = plsc.VectorSubcoreMesh(core_axis_name="c", subcore_axis_name="s")

    @functools.partial(
        pl.kernel,
        mesh=mesh,
        out_type=jax.ShapeDtypeStruct((B, S, D), jnp.float32),
        scratch_types=[
            pltpu.VMEM((rows_per_w,), jnp.int32),
            pltpu.VMEM((2, S, 128), jnp.float32),
            pltpu.VMEM((2, S, D), jnp.float32),
            pltpu.SemaphoreType.DMA((2,)),
            pltpu.SemaphoreType.DMA((2,)),
        ],
    )
    def gather_kernel(idx_hbm, table_hbm, out_hbm, idx_v, rows_v, rows_c,
                      sem_g, sem_w):
        wid = lax.axis_index("s") * NC + lax.axis_index("c")
        base = wid * rows_per_w
        slab0 = wid * slabs_per_w

        pltpu.sync_copy(idx_hbm.at[pl.ds(base, rows_per_w)], idx_v)

        def fire_gather(i, p):
            off = i * S
            pltpu.async_copy(
                table_hbm.at[idx_v.at[pl.ds(off, G0)]],
                rows_v.at[p, pl.ds(0, G0), :],
                sem_g.at[p],
            )
            pltpu.async_copy(
                table_hbm.at[idx_v.at[pl.ds(off + G0, G1)]],
                rows_v.at[p, pl.ds(G0, G1), :],
                sem_g.at[p],
            )

        def wait_gather(p):
            pltpu.make_async_copy(
                table_hbm.at[idx_v.at[pl.ds(0, G0)]],
                rows_v.at[p, pl.ds(0, G0), :],
                sem_g.at[p],
            ).wait()
            pltpu.make_async_copy(
                table_hbm.at[idx_v.at[pl.ds(0, G1)]],
                rows_v.at[p, pl.ds(G0, G1), :],
                sem_g.at[p],
            ).wait()

        def wait_write(p):
            pltpu.make_async_copy(
                rows_c.at[p], out_hbm.at[slab0], sem_w.at[p]
            ).wait()

        fire_gather(0, 0)

        def body(g, carry):
            for p in (0, 1):
                i = 2 * g + p
                np_ = 1 - p

                @pl.when(i + 1 < slabs_per_w)
                def _():
                    fire_gather(i + 1, np_)

                wait_gather(p)

                @pl.when(i >= 2)
                def _():
                    wait_write(p)

                def compact(q, c2):
                    for u in range(4):
                        r = 4 * q + u
                        for k in range(D // 16):
                            rows_c[p, r, pl.ds(k * 16, 16)] = (
                                rows_v[p, r, pl.ds(k * 16, 16)])
                    return c2

                lax.fori_loop(0, S // 4, compact, 0)
                pltpu.async_copy(
                    rows_c.at[p], out_hbm.at[slab0 + i], sem_w.at[p]
                )
            return carry

        lax.fori_loop(0, slabs_per_w // 2, body, 0)
        wait_write(0)
        wait_write(1)

    return gather_kernel(idx_flat, table_w)


# speedup vs baseline: 1.1235x; 1.0022x over previous
import functools

import jax
import jax.numpy as jnp
from jax import lax
from jax.experimental import pallas as pl
from jax.experimental.pallas import tpu as pltpu
from jax.experimental.pallas import tpu_sc as plsc


def kernel(inputs, table):
    B, S = inputs.shape
    V, D = table.shape
    table_w = jnp.pad(table, ((0, 0), (0, 128 - D)))

    info = plsc.get_sparse_core_info()
    NC, NS = info.num_cores, info.num_subcores
    NW = NC * NS
    slabs_per_w = B // NW
    HS = slabs_per_w // 2
    G0 = 128
    G1 = S - G0

    mesh = plsc.VectorSubcoreMesh(core_axis_name="c", subcore_axis_name="s")

    @functools.partial(
        pl.kernel,
        mesh=mesh,
        out_type=jax.ShapeDtypeStruct((B, S, D), jnp.float32),
        scratch_types=[
            pltpu.VMEM((HS, G0), jnp.int32),
            pltpu.VMEM((HS, G1), jnp.int32),
            pltpu.VMEM((2, S, 128), jnp.float32),
            pltpu.VMEM((2, S, D), jnp.float32),
            pltpu.SemaphoreType.DMA((2,)),
            pltpu.SemaphoreType.DMA((2,)),
        ],
    )
    def gather_kernel(idx_hbm, table_hbm, out_hbm, idx_a, idx_b, rows_v,
                      rows_c, sem_g, sem_w):
        wid = lax.axis_index("s") * NC + lax.axis_index("c")
        slab0 = wid * slabs_per_w

        def load_idx(half):
            b0 = slab0 + half * HS
            pltpu.sync_copy(idx_hbm.at[pl.ds(b0, HS), pl.ds(0, G0)], idx_a)
            pltpu.sync_copy(idx_hbm.at[pl.ds(b0, HS), pl.ds(G0, G1)], idx_b)

        def fire_gather(r, p):
            pltpu.async_copy(
                table_hbm.at[idx_a.at[r]],
                rows_v.at[p, pl.ds(0, G0), :],
                sem_g.at[p],
            )
            pltpu.async_copy(
                table_hbm.at[idx_b.at[r]],
                rows_v.at[p, pl.ds(G0, G1), :],
                sem_g.at[p],
            )

        def wait_gather(p):
            pltpu.make_async_copy(
                table_hbm.at[idx_a.at[0]],
                rows_v.at[p, pl.ds(0, G0), :],
                sem_g.at[p],
            ).wait()
            pltpu.make_async_copy(
                table_hbm.at[idx_b.at[0]],
                rows_v.at[p, pl.ds(G0, G1), :],
                sem_g.at[p],
            ).wait()

        def wait_write(p):
            pltpu.make_async_copy(
                rows_c.at[p], out_hbm.at[slab0], sem_w.at[p]
            ).wait()

        for half in (0, 1):
            load_idx(half)
            fire_gather(0, 0)

            def body(g, carry):
                for p in (0, 1):
                    i = 2 * g + p
                    np_ = 1 - p

                    @pl.when(i + 1 < HS)
                    def _():
                        fire_gather(i + 1, np_)

                    wait_gather(p)

                    @pl.when((half > 0) | (i >= 2))
                    def _():
                        wait_write(p)

                    def compact(q, c2):
                        for u in range(4):
                            r = 4 * q + u
                            for k in range(D // 16):
                                rows_c[p, r, pl.ds(k * 16, 16)] = (
                                    rows_v[p, r, pl.ds(k * 16, 16)])
                        return c2

                    lax.fori_loop(0, S // 4, compact, 0)
                    pltpu.async_copy(
                        rows_c.at[p],
                        out_hbm.at[slab0 + half * HS + i],
                        sem_w.at[p],
                    )
                return carry

            lax.fori_loop(0, HS // 2, body, 0)
        wait_write(0)
        wait_write(1)

    return gather_kernel(inputs, table_w)
